# baseline (device time: 15417 ns/iter reference)
import jax
import jax.numpy as jnp
from jax import lax
from jax.experimental import pallas as pl
from jax.experimental.pallas import tpu as pltpu

N_DEV = 4


def kernel(x, w_mat):
    m_total, k_shard = x.shape
    k_total, n = w_mat.shape
    m_per = m_total // N_DEV

    def body(x_ref, w_hbm, out_ref, stage_ref, w_buf, w_sems):
        my = lax.axis_index("i")

        def w_fetch(j, slot):
            cp = pltpu.make_async_copy(
                w_hbm.at[pl.ds(j * k_shard, k_shard), :],
                w_buf.at[slot],
                w_sems.at[slot],
            )
            cp.start()
            return cp

        w_cps = [w_fetch(my, 0)]

        for d in range(1, N_DEV):
            peer = lax.rem(my + d, N_DEV)
            stage_ref[d] = x_ref[pl.ds(peer * m_per, m_per), :].astype(
                jnp.bfloat16
            )

        w_cps[0].wait()
        w_bf16 = w_buf[0].astype(jnp.bfloat16)
        out_ref[...] = jnp.dot(
            x_ref[pl.ds(my * m_per, m_per), :].astype(jnp.bfloat16),
            w_bf16,
            preferred_element_type=jnp.float32,
        )

        for d in range(1, N_DEV):
            out_ref[...] += jnp.dot(
                stage_ref[d],
                w_bf16,
                preferred_element_type=jnp.float32,
            )

        out_ref[...] = jnp.maximum(out_ref[...], 0.0)

    return pl.pallas_call(
        body,
        out_shape=jax.ShapeDtypeStruct((m_per, n), jnp.float32),
        in_specs=[
            pl.BlockSpec(memory_space=pltpu.VMEM),
            pl.BlockSpec(memory_space=pl.ANY),
        ],
        out_specs=pl.BlockSpec(memory_space=pltpu.VMEM),
        scratch_shapes=[
            pltpu.VMEM((N_DEV, m_per, k_shard), jnp.bfloat16),
            pltpu.VMEM((2, k_shard, n), jnp.float32),
            pltpu.SemaphoreType.DMA((2,)),
        ],
    )(x, w_mat)
